# baseline (speedup 1.0000x reference)
import functools

import jax
import jax.numpy as jnp
from jax import lax
from jax.experimental import pallas as pl
from jax.experimental.pallas import tpu as pltpu
from jax.experimental.pallas import tpu_sc as plsc

N = 10000
E = 320000
D_IN = 128
H1 = 256
H2 = 32
C = 7
CPAD = 16

NSC = 2
NTL = 16
NW = NSC * NTL
EPT = E // NW
NP = 10240
RPT = NP // NTL

CH1, NCH1 = 48, 216
CH2, NCH2 = 128, 80

_MESH = dict(core_axis_name="c", subcore_axis_name="s",
             num_cores=NSC, num_subcores=NTL)


def _fill(ref, rows, width, value):
    vals = jnp.full((16,), value, jnp.float32)

    def body(i, _):
        for j in range(width // 16):
            ref[i, pl.ds(j * 16, 16)] = vals
        return 0

    lax.fori_loop(0, rows, body, 0)


def _deg_body(dst_hbm, out_hbm, acc, idx_d, ones, s0, s1):
    cid = lax.axis_index("c")
    sid = lax.axis_index("s")
    tid = cid * NTL + sid
    _fill(ones, CH2, 16, 0.0)
    for i in range(RPT // CH2):
        pltpu.sync_copy(ones, acc.at[pl.ds(sid * RPT + i * CH2, CH2)])
    _fill(ones, CH2, 16, 1.0)
    pltpu.sync_copy(dst_hbm.at[tid], idx_d)
    plsc.subcore_barrier()

    ssem = (s0, s1)

    def scat(c, b):
        pltpu.async_copy(ones, acc.at[idx_d.at[c]], ssem[b], add=True)

    def wait_s(b):
        pltpu.make_async_copy(ones, acc.at[idx_d.at[0]], ssem[b]).wait()

    scat(0, 0)
    scat(1, 1)

    def pair(q, _):
        c = 2 * q
        wait_s(0)
        scat(c, 0)
        wait_s(1)
        scat(c + 1, 1)
        return 0

    lax.fori_loop(1, NCH2 // 2 - 1, pair, 0)
    wait_s(0)
    pltpu.sync_copy(ones, acc.at[idx_d.at[NCH2 - 2]], add=True)
    wait_s(1)
    pltpu.sync_copy(ones, acc.at[idx_d.at[NCH2 - 1]], add=True)
    plsc.subcore_barrier()
    pltpu.sync_copy(acc.at[pl.ds(sid * RPT, RPT)],
                    out_hbm.at[cid, pl.ds(sid * RPT, RPT)])


_deg_kernel = functools.partial(
    pl.kernel,
    out_type=jax.ShapeDtypeStruct((NSC, NP, 16), jnp.float32),
    mesh=plsc.VectorSubcoreMesh(**_MESH),
    compiler_params=pltpu.CompilerParams(use_tc_tiling_on_sc=False),
    scratch_types=[
        pltpu.VMEM_SHARED((NP, 16), jnp.float32),
        pltpu.VMEM((NCH2, CH2), jnp.int32),
        pltpu.VMEM((CH2, 16), jnp.float32),
        pltpu.SemaphoreType.DMA,
        pltpu.SemaphoreType.DMA,
    ],
)(_deg_body)


def _make_agg(F, CH, NCH):
    full, rem = divmod(RPT, CH)

    def body(src_hbm, dst_hbm, xs_hbm, out_hbm, acc, idx_s, idx_d,
             r0, r1, r2, r3, g0, g1, g2, g3, s0, s1, s2, s3):
        rows = (r0, r1, r2, r3)
        gsem = (g0, g1, g2, g3)
        ssem = (s0, s1, s2, s3)
        cid = lax.axis_index("c")
        sid = lax.axis_index("s")
        tid = cid * NTL + sid

        def gath(c, b):
            pltpu.async_copy(xs_hbm.at[idx_s.at[c]], rows[b], gsem[b])

        def wait_g(c, b):
            pltpu.make_async_copy(xs_hbm.at[idx_s.at[c]], rows[b],
                                  gsem[b]).wait()

        def scat(c, b):
            pltpu.async_copy(rows[b], acc.at[idx_d.at[c]], ssem[b], add=True)

        def scat_sync(c, b):
            pltpu.sync_copy(rows[b], acc.at[idx_d.at[c]], add=True)

        def wait_s(b):
            pltpu.make_async_copy(rows[b], acc.at[idx_d.at[0]],
                                  ssem[b]).wait()

        _fill(r0, CH, F, 0.0)
        for i in range(full):
            pltpu.sync_copy(r0, acc.at[pl.ds(sid * RPT + i * CH, CH)])
        if rem:
            pltpu.sync_copy(r0.at[pl.ds(0, rem)],
                            acc.at[pl.ds(sid * RPT + full * CH, rem)])
        pltpu.sync_copy(src_hbm.at[tid], idx_s)
        pltpu.sync_copy(dst_hbm.at[tid], idx_d)
        plsc.subcore_barrier()

        gath(0, 0)
        gath(1, 1)
        wait_g(0, 0); scat(0, 0); gath(2, 2)
        wait_g(1, 1); scat(1, 1); gath(3, 3)
        wait_g(2, 2); scat(2, 2); wait_s(0); gath(4, 0)
        wait_g(3, 3); scat(3, 3); wait_s(1); gath(5, 1)

        def round_body(q, _):
            c0 = q * 4
            for b in range(4):
                c = c0 + b
                wait_g(c, b)
                scat(c, b)
                wait_s((b + 2) % 4)
                gath(c + 2, (b + 2) % 4)
            return 0

        lax.fori_loop(1, NCH // 4 - 1, round_body, 0)
        T = NCH - 4
        wait_g(T, 0); scat(T, 0); wait_s(2); gath(T + 2, 2)
        wait_g(T + 1, 1); scat(T + 1, 1); wait_s(3); gath(T + 3, 3)
        wait_g(T + 2, 2); wait_s(0); scat_sync(T + 2, 2)
        wait_g(T + 3, 3); wait_s(1); scat_sync(T + 3, 3)
        plsc.subcore_barrier()
        pltpu.sync_copy(acc.at[pl.ds(sid * RPT, RPT)],
                        out_hbm.at[cid, pl.ds(sid * RPT, RPT)])

    return functools.partial(
        pl.kernel,
        out_type=jax.ShapeDtypeStruct((NSC, NP, F), jnp.float32),
        mesh=plsc.VectorSubcoreMesh(**_MESH),
        compiler_params=pltpu.CompilerParams(use_tc_tiling_on_sc=False),
        scratch_types=(
            [pltpu.VMEM_SHARED((NP, F), jnp.float32),
             pltpu.VMEM((NCH, CH), jnp.int32),
             pltpu.VMEM((NCH, CH), jnp.int32)]
            + [pltpu.VMEM((CH, F), jnp.float32) for _ in range(4)]
            + [pltpu.SemaphoreType.DMA for _ in range(8)]
        ),
    )(body)


_agg128 = _make_agg(D_IN, CH1, NCH1)
_agg32 = _make_agg(H2, CH2, NCH2)
_agg16 = _make_agg(CPAD, CH2, NCH2)


def _tc1_body(degp_ref, x_ref, dis_ref, xs_ref):
    degp = degp_ref[...]
    deg = degp[0, :N, 0:1] + degp[1, :N, 0:1] + 1.0
    dis = lax.rsqrt(deg)
    dis_ref[...] = dis
    xs_ref[...] = x_ref[...] * dis


_tc1 = pl.pallas_call(
    _tc1_body,
    out_shape=[
        jax.ShapeDtypeStruct((N, 1), jnp.float32),
        jax.ShapeDtypeStruct((N, D_IN), jnp.float32),
    ],
)


def _tc2_body(p_ref, xs_ref, dis_ref, W1_ref, b1_ref, W2_ref, t2s_ref):
    p = p_ref[...]
    dis = dis_ref[...]
    agg1 = (p[0, :N] + p[1, :N] + xs_ref[...]) * dis
    h1 = jnp.dot(agg1, W1_ref[...], preferred_element_type=jnp.float32)
    h1 = jnp.maximum(h1 + b1_ref[...][None, :], 0.0)
    t2 = jnp.dot(h1, W2_ref[...], preferred_element_type=jnp.float32)
    t2s_ref[...] = t2 * dis


_tc2 = pl.pallas_call(
    _tc2_body,
    out_shape=jax.ShapeDtypeStruct((N, H2), jnp.float32),
)


def _tc3_body(q_ref, t2s_ref, dis_ref, b2_ref, W3p_ref, t3s_ref):
    q = q_ref[...]
    dis = dis_ref[...]
    agg2 = (q[0, :N] + q[1, :N] + t2s_ref[...]) * dis
    h2 = jnp.maximum(agg2 + b2_ref[...][None, :], 0.0)
    t3 = jnp.dot(h2, W3p_ref[...], preferred_element_type=jnp.float32)
    t3s_ref[...] = t3 * dis


_tc3 = pl.pallas_call(
    _tc3_body,
    out_shape=jax.ShapeDtypeStruct((N, CPAD), jnp.float32),
)


def _tc4_body(r_ref, t3s_ref, dis_ref, b3_ref, out_ref):
    r = r_ref[...]
    h3p = (r[0, :N] + r[1, :N] + t3s_ref[...]) * dis_ref[...]
    h3 = h3p[:, :C] + b3_ref[...][None, :]
    m = jnp.max(h3, axis=1, keepdims=True)
    e = jnp.exp(h3 - m)
    lse = jnp.log(jnp.sum(e, axis=1, keepdims=True)) + m
    out_ref[...] = h3 - lse


_tc4 = pl.pallas_call(
    _tc4_body,
    out_shape=jax.ShapeDtypeStruct((N, C), jnp.float32),
)


def _pad_edges(v, nch, ch, fill):
    total = NW * nch * ch
    pad = jnp.full((total - E,), fill, jnp.int32)
    return jnp.concatenate([v, pad]).reshape(NW, nch, ch)


def kernel(x, edge_index, W1, b1, W2, b2, W3, b3):
    src1 = _pad_edges(edge_index[0], NCH1, CH1, 0)
    dst1 = _pad_edges(edge_index[1], NCH1, CH1, NP - 1)
    src2 = _pad_edges(edge_index[0], NCH2, CH2, 0)
    dst2 = _pad_edges(edge_index[1], NCH2, CH2, NP - 1)
    W3p = jnp.pad(W3, ((0, 0), (0, CPAD - C)))

    degp = _deg_kernel(dst2)
    dis, xs = _tc1(degp, x)
    p1 = _agg128(src1, dst1, xs)
    t2s = _tc2(p1, xs, dis, W1, b1, W2)
    p2 = _agg32(src2, dst2, t2s)
    t3s = _tc3(p2, t2s, dis, b2, W3p)
    p3 = _agg16(src2, dst2, t3s)
    return _tc4(p3, t3s, dis, b3)


# speedup vs baseline: 2.3033x; 2.3033x over previous
import functools

import jax
import jax.numpy as jnp
from jax import lax
from jax.experimental import pallas as pl
from jax.experimental.pallas import tpu as pltpu
from jax.experimental.pallas import tpu_sc as plsc

N = 10000
E = 320000
D_IN = 128
H1 = 256
H2 = 32
C = 7
CPAD = 16

NSC = 2
NTL = 16
NW = NSC * NTL
EPT = E // NW
NP = 10240
RPT = NP // NTL

CH1, NCH1 = 48, 216
CH2, NCH2 = 128, 80

_MESH = dict(core_axis_name="c", subcore_axis_name="s",
             num_cores=NSC, num_subcores=NTL)


def _fill(ref, rows, width, value):
    vals = jnp.full((16,), value, jnp.float32)

    def body(i, _):
        for j in range(width // 16):
            ref[i, pl.ds(j * 16, 16)] = vals
        return 0

    lax.fori_loop(0, rows, body, 0)


def _deg_body(dst_hbm, out_hbm, acc, idx_d, ones, s0, s1):
    cid = lax.axis_index("c")
    sid = lax.axis_index("s")
    tid = cid * NTL + sid
    _fill(ones, CH2, 16, 0.0)
    for i in range(RPT // CH2):
        pltpu.sync_copy(ones, acc.at[pl.ds(sid * RPT + i * CH2, CH2)])
    _fill(ones, CH2, 16, 1.0)
    pltpu.sync_copy(dst_hbm.at[tid], idx_d)
    plsc.subcore_barrier()

    ssem = (s0, s1)

    def scat(c, b):
        pltpu.async_copy(ones, acc.at[idx_d.at[c]], ssem[b], add=True)

    def wait_s(b):
        pltpu.make_async_copy(ones, acc.at[idx_d.at[0]], ssem[b]).wait()

    scat(0, 0)
    scat(1, 1)

    def pair(q, _):
        c = 2 * q
        wait_s(0)
        scat(c, 0)
        wait_s(1)
        scat(c + 1, 1)
        return 0

    lax.fori_loop(1, NCH2 // 2 - 1, pair, 0)
    wait_s(0)
    pltpu.sync_copy(ones, acc.at[idx_d.at[NCH2 - 2]], add=True)
    wait_s(1)
    pltpu.sync_copy(ones, acc.at[idx_d.at[NCH2 - 1]], add=True)
    plsc.subcore_barrier()
    pltpu.sync_copy(acc.at[pl.ds(sid * RPT, RPT)],
                    out_hbm.at[cid, pl.ds(sid * RPT, RPT)])


_deg_kernel = functools.partial(
    pl.kernel,
    out_type=jax.ShapeDtypeStruct((NSC, NP, 16), jnp.float32),
    mesh=plsc.VectorSubcoreMesh(**_MESH),
    compiler_params=pltpu.CompilerParams(use_tc_tiling_on_sc=False),
    scratch_types=[
        pltpu.VMEM_SHARED((NP, 16), jnp.float32),
        pltpu.VMEM((NCH2, CH2), jnp.int32),
        pltpu.VMEM((CH2, 16), jnp.float32),
        pltpu.SemaphoreType.DMA,
        pltpu.SemaphoreType.DMA,
    ],
)(_deg_body)


def _make_agg(F, CH, NCH):
    full, rem = divmod(RPT, CH)

    def body(src_hbm, dst_hbm, xs_hbm, out_hbm,
             acc, idx_s, idx_d, rows_a, rows_b, sem_a, sem_b):
        cid = lax.axis_index("c")
        sid = lax.axis_index("s")
        tid = cid * NTL + sid
        _fill(rows_a, CH, F, 0.0)
        for i in range(full):
            pltpu.sync_copy(rows_a, acc.at[pl.ds(sid * RPT + i * CH, CH)])
        if rem:
            pltpu.sync_copy(rows_a.at[pl.ds(0, rem)],
                            acc.at[pl.ds(sid * RPT + full * CH, rem)])
        pltpu.sync_copy(src_hbm.at[tid], idx_s)
        pltpu.sync_copy(dst_hbm.at[tid], idx_d)
        plsc.subcore_barrier()

        pltpu.async_copy(xs_hbm.at[idx_s.at[0]], rows_a, sem_a)
        pltpu.async_copy(xs_hbm.at[idx_s.at[1]], rows_b, sem_b)

        def pair(i, _):
            c = 2 * i
            pltpu.make_async_copy(xs_hbm.at[idx_s.at[c]], rows_a, sem_a).wait()
            pltpu.sync_copy(rows_a, acc.at[idx_d.at[c]], add=True)
            na = jnp.minimum(c + 2, NCH - 1)
            pltpu.async_copy(xs_hbm.at[idx_s.at[na]], rows_a, sem_a)
            pltpu.make_async_copy(xs_hbm.at[idx_s.at[c + 1]], rows_b, sem_b).wait()
            pltpu.sync_copy(rows_b, acc.at[idx_d.at[c + 1]], add=True)
            nb = jnp.minimum(c + 3, NCH - 1)
            pltpu.async_copy(xs_hbm.at[idx_s.at[nb]], rows_b, sem_b)
            return 0

        lax.fori_loop(0, NCH // 2, pair, 0)
        pltpu.make_async_copy(xs_hbm.at[idx_s.at[0]], rows_a, sem_a).wait()
        pltpu.make_async_copy(xs_hbm.at[idx_s.at[0]], rows_b, sem_b).wait()
        plsc.subcore_barrier()
        pltpu.sync_copy(acc.at[pl.ds(sid * RPT, RPT)],
                        out_hbm.at[cid, pl.ds(sid * RPT, RPT)])

    return functools.partial(
        pl.kernel,
        out_type=jax.ShapeDtypeStruct((NSC, NP, F), jnp.float32),
        mesh=plsc.VectorSubcoreMesh(**_MESH),
        compiler_params=pltpu.CompilerParams(use_tc_tiling_on_sc=False),
        scratch_types=[
            pltpu.VMEM_SHARED((NP, F), jnp.float32),
            pltpu.VMEM((NCH, CH), jnp.int32),
            pltpu.VMEM((NCH, CH), jnp.int32),
            pltpu.VMEM((CH, F), jnp.float32),
            pltpu.VMEM((CH, F), jnp.float32),
            pltpu.SemaphoreType.DMA,
            pltpu.SemaphoreType.DMA,
        ],
    )(body)


_agg128 = _make_agg(D_IN, 100, 100)
_agg32 = _make_agg(H2, CH2, NCH2)
_agg16 = _make_agg(CPAD, CH2, NCH2)


def _tc1_body(degp_ref, x_ref, dis_ref, xs_ref):
    degp = degp_ref[...]
    deg = degp[0, :N, 0:1] + degp[1, :N, 0:1] + 1.0
    dis = lax.rsqrt(deg)
    dis_ref[...] = dis
    xs_ref[...] = x_ref[...] * dis


_tc1 = pl.pallas_call(
    _tc1_body,
    out_shape=[
        jax.ShapeDtypeStruct((N, 1), jnp.float32),
        jax.ShapeDtypeStruct((N, D_IN), jnp.float32),
    ],
)


def _tc2_body(p_ref, xs_ref, dis_ref, W1_ref, b1_ref, W2_ref, t2s_ref):
    p = p_ref[...]
    dis = dis_ref[...]
    agg1 = (p[0, :N] + p[1, :N] + xs_ref[...]) * dis
    h1 = jnp.dot(agg1, W1_ref[...], preferred_element_type=jnp.float32)
    h1 = jnp.maximum(h1 + b1_ref[...][None, :], 0.0)
    t2 = jnp.dot(h1, W2_ref[...], preferred_element_type=jnp.float32)
    t2s_ref[...] = t2 * dis


_tc2 = pl.pallas_call(
    _tc2_body,
    out_shape=jax.ShapeDtypeStruct((N, H2), jnp.float32),
)


def _tc3_body(q_ref, t2s_ref, dis_ref, b2_ref, W3p_ref, t3s_ref):
    q = q_ref[...]
    dis = dis_ref[...]
    agg2 = (q[0, :N] + q[1, :N] + t2s_ref[...]) * dis
    h2 = jnp.maximum(agg2 + b2_ref[...][None, :], 0.0)
    t3 = jnp.dot(h2, W3p_ref[...], preferred_element_type=jnp.float32)
    t3s_ref[...] = t3 * dis


_tc3 = pl.pallas_call(
    _tc3_body,
    out_shape=jax.ShapeDtypeStruct((N, CPAD), jnp.float32),
)


def _tc4_body(r_ref, t3s_ref, dis_ref, b3_ref, out_ref):
    r = r_ref[...]
    h3p = (r[0, :N] + r[1, :N] + t3s_ref[...]) * dis_ref[...]
    h3 = h3p[:, :C] + b3_ref[...][None, :]
    m = jnp.max(h3, axis=1, keepdims=True)
    e = jnp.exp(h3 - m)
    lse = jnp.log(jnp.sum(e, axis=1, keepdims=True)) + m
    out_ref[...] = h3 - lse


_tc4 = pl.pallas_call(
    _tc4_body,
    out_shape=jax.ShapeDtypeStruct((N, C), jnp.float32),
)


def _pad_edges(v, nch, ch, fill):
    total = NW * nch * ch
    pad = jnp.full((total - E,), fill, jnp.int32)
    return jnp.concatenate([v, pad]).reshape(NW, nch, ch)


def kernel(x, edge_index, W1, b1, W2, b2, W3, b3):
    src1 = edge_index[0].reshape(NW, 100, 100)
    dst1 = edge_index[1].reshape(NW, 100, 100)
    src2 = _pad_edges(edge_index[0], NCH2, CH2, 0)
    dst2 = _pad_edges(edge_index[1], NCH2, CH2, NP - 1)
    W3p = jnp.pad(W3, ((0, 0), (0, CPAD - C)))

    degp = _deg_kernel(dst2)
    dis, xs = _tc1(degp, x)
    p1 = _agg128(src1, dst1, xs)
    t2s = _tc2(p1, xs, dis, W1, b1, W2)
    p2 = _agg32(src2, dst2, t2s)
    t3s = _tc3(p2, t2s, dis, b2, W3p)
    p3 = _agg16(src2, dst2, t3s)
    return _tc4(p3, t3s, dis, b3)


# speedup vs baseline: 3.1274x; 1.3578x over previous
import functools

import jax
import jax.numpy as jnp
from jax import lax
from jax.experimental import pallas as pl
from jax.experimental.pallas import tpu as pltpu
from jax.experimental.pallas import tpu_sc as plsc

N = 10000
E = 320000
D_IN = 128
H1 = 256
H2 = 32
C = 7
CPAD = 16

NSC = 2
NTL = 16
NW = NSC * NTL
EPT = E // NW
NP = 10240
RPT = NP // NTL

CH1, NCH1 = 48, 216
CH2, NCH2 = 128, 80

_MESH = dict(core_axis_name="c", subcore_axis_name="s",
             num_cores=NSC, num_subcores=NTL)


def _fill(ref, rows, width, value):
    vals = jnp.full((16,), value, jnp.float32)

    def body(i, _):
        for j in range(width // 16):
            ref[i, pl.ds(j * 16, 16)] = vals
        return 0

    lax.fori_loop(0, rows, body, 0)


def _deg_body(dst_hbm, out_hbm, acc, idx_d, ones, s0, s1):
    cid = lax.axis_index("c")
    sid = lax.axis_index("s")
    tid = cid * NTL + sid
    _fill(ones, CH2, 16, 0.0)
    for i in range(RPT // CH2):
        pltpu.sync_copy(ones, acc.at[pl.ds(sid * RPT + i * CH2, CH2)])
    _fill(ones, CH2, 16, 1.0)
    pltpu.sync_copy(dst_hbm.at[tid], idx_d)
    plsc.subcore_barrier()

    ssem = (s0, s1)

    def scat(c, b):
        pltpu.async_copy(ones, acc.at[idx_d.at[c]], ssem[b], add=True)

    def wait_s(b):
        pltpu.make_async_copy(ones, acc.at[idx_d.at[0]], ssem[b]).wait()

    scat(0, 0)
    scat(1, 1)

    def pair(q, _):
        c = 2 * q
        wait_s(0)
        scat(c, 0)
        wait_s(1)
        scat(c + 1, 1)
        return 0

    lax.fori_loop(1, NCH2 // 2 - 1, pair, 0)
    wait_s(0)
    pltpu.sync_copy(ones, acc.at[idx_d.at[NCH2 - 2]], add=True)
    wait_s(1)
    pltpu.sync_copy(ones, acc.at[idx_d.at[NCH2 - 1]], add=True)
    plsc.subcore_barrier()
    pltpu.sync_copy(acc.at[pl.ds(sid * RPT, RPT)],
                    out_hbm.at[cid, pl.ds(sid * RPT, RPT)])


_deg_kernel = functools.partial(
    pl.kernel,
    out_type=jax.ShapeDtypeStruct((NSC, NP, 16), jnp.float32),
    mesh=plsc.VectorSubcoreMesh(**_MESH),
    compiler_params=pltpu.CompilerParams(use_tc_tiling_on_sc=False),
    scratch_types=[
        pltpu.VMEM_SHARED((NP, 16), jnp.float32),
        pltpu.VMEM((NCH2, CH2), jnp.int32),
        pltpu.VMEM((CH2, 16), jnp.float32),
        pltpu.SemaphoreType.DMA,
        pltpu.SemaphoreType.DMA,
    ],
)(_deg_body)


def _make_agg(F, CH, NCH):
    full, rem = divmod(RPT, CH)

    def body(src_hbm, dst_hbm, xs_hbm, out_hbm,
             acc, idx_s, idx_d, rows_a, rows_b, sem_a, sem_b):
        cid = lax.axis_index("c")
        sid = lax.axis_index("s")
        tid = cid * NTL + sid
        _fill(rows_a, CH, F, 0.0)
        for i in range(full):
            pltpu.sync_copy(rows_a, acc.at[pl.ds(sid * RPT + i * CH, CH)])
        if rem:
            pltpu.sync_copy(rows_a.at[pl.ds(0, rem)],
                            acc.at[pl.ds(sid * RPT + full * CH, rem)])
        pltpu.sync_copy(src_hbm.at[tid], idx_s)
        pltpu.sync_copy(dst_hbm.at[tid], idx_d)
        plsc.subcore_barrier()

        pltpu.async_copy(xs_hbm.at[idx_s.at[0]], rows_a, sem_a)
        pltpu.async_copy(xs_hbm.at[idx_s.at[1]], rows_b, sem_b)

        def pair(i, _):
            c = 2 * i
            pltpu.make_async_copy(xs_hbm.at[idx_s.at[c]], rows_a, sem_a).wait()
            pltpu.sync_copy(rows_a, acc.at[idx_d.at[c]], add=True)
            na = jnp.minimum(c + 2, NCH - 1)
            pltpu.async_copy(xs_hbm.at[idx_s.at[na]], rows_a, sem_a)
            pltpu.make_async_copy(xs_hbm.at[idx_s.at[c + 1]], rows_b, sem_b).wait()
            pltpu.sync_copy(rows_b, acc.at[idx_d.at[c + 1]], add=True)
            nb = jnp.minimum(c + 3, NCH - 1)
            pltpu.async_copy(xs_hbm.at[idx_s.at[nb]], rows_b, sem_b)
            return 0

        lax.fori_loop(0, NCH // 2, pair, 0)
        pltpu.make_async_copy(xs_hbm.at[idx_s.at[0]], rows_a, sem_a).wait()
        pltpu.make_async_copy(xs_hbm.at[idx_s.at[0]], rows_b, sem_b).wait()
        plsc.subcore_barrier()
        pltpu.sync_copy(acc.at[pl.ds(sid * RPT, RPT)],
                        out_hbm.at[cid, pl.ds(sid * RPT, RPT)])

    return functools.partial(
        pl.kernel,
        out_type=jax.ShapeDtypeStruct((NSC, NP, F), jnp.float32),
        mesh=plsc.VectorSubcoreMesh(**_MESH),
        compiler_params=pltpu.CompilerParams(use_tc_tiling_on_sc=False),
        scratch_types=[
            pltpu.VMEM_SHARED((NP, F), jnp.float32),
            pltpu.VMEM((NCH, CH), jnp.int32),
            pltpu.VMEM((NCH, CH), jnp.int32),
            pltpu.VMEM((CH, F), jnp.float32),
            pltpu.VMEM((CH, F), jnp.float32),
            pltpu.SemaphoreType.DMA,
            pltpu.SemaphoreType.DMA,
        ],
    )(body)


_agg128 = _make_agg(D_IN, 100, 100)
_agg32 = _make_agg(H2, CH2, NCH2)
_agg16 = _make_agg(CPAD, CH2, NCH2)


def _tc1_body(degp_ref, x_ref, dis_ref, xs_ref):
    degp = degp_ref[...]
    deg = degp[0, :N, 0:1] + degp[1, :N, 0:1] + 1.0
    dis = lax.rsqrt(deg)
    dis_ref[...] = dis
    xs_ref[...] = x_ref[...] * dis


_tc1 = pl.pallas_call(
    _tc1_body,
    out_shape=[
        jax.ShapeDtypeStruct((N, 1), jnp.float32),
        jax.ShapeDtypeStruct((N, D_IN), jnp.float32),
    ],
)


def _tc2_body(p_ref, xs_ref, dis_ref, W1_ref, b1_ref, W2_ref, t2s_ref):
    p = p_ref[...]
    dis = dis_ref[...]
    agg1 = (p[0, :N] + p[1, :N] + xs_ref[...]) * dis
    h1 = jnp.dot(agg1, W1_ref[...], preferred_element_type=jnp.float32)
    h1 = jnp.maximum(h1 + b1_ref[...][None, :], 0.0)
    t2 = jnp.dot(h1, W2_ref[...], preferred_element_type=jnp.float32)
    t2s_ref[...] = t2 * dis


_tc2 = pl.pallas_call(
    _tc2_body,
    out_shape=jax.ShapeDtypeStruct((N, H2), jnp.float32),
)


def _tc3_body(q_ref, t2s_ref, dis_ref, b2_ref, W3p_ref, t3s_ref):
    q = q_ref[...]
    dis = dis_ref[...]
    agg2 = (q[0, :N] + q[1, :N] + t2s_ref[...]) * dis
    h2 = jnp.maximum(agg2 + b2_ref[...][None, :], 0.0)
    t3 = jnp.dot(h2, W3p_ref[...], preferred_element_type=jnp.float32)
    t3s_ref[...] = t3 * dis


_tc3 = pl.pallas_call(
    _tc3_body,
    out_shape=jax.ShapeDtypeStruct((N, CPAD), jnp.float32),
)


def _tc4_body(r_ref, t3s_ref, dis_ref, b3_ref, out_ref):
    r = r_ref[...]
    h3p = (r[0, :N] + r[1, :N] + t3s_ref[...]) * dis_ref[...]
    h3 = h3p[:, :C] + b3_ref[...][None, :]
    m = jnp.max(h3, axis=1, keepdims=True)
    e = jnp.exp(h3 - m)
    lse = jnp.log(jnp.sum(e, axis=1, keepdims=True)) + m
    out_ref[...] = h3 - lse


_tc4 = pl.pallas_call(
    _tc4_body,
    out_shape=jax.ShapeDtypeStruct((N, C), jnp.float32),
)


def _pad_edges(v, nch, ch, dummy_dst):
    total = NW * nch * ch
    pad = jnp.arange(total - E, dtype=jnp.int32)
    pad = N + pad % (NP - N) if dummy_dst else pad % N
    return jnp.concatenate([v, pad]).reshape(NW, nch, ch)


def kernel(x, edge_index, W1, b1, W2, b2, W3, b3):
    src1 = edge_index[0].reshape(NW, 100, 100)
    dst1 = edge_index[1].reshape(NW, 100, 100)
    src2 = _pad_edges(edge_index[0], NCH2, CH2, False)
    dst2 = _pad_edges(edge_index[1], NCH2, CH2, True)
    W3p = jnp.pad(W3, ((0, 0), (0, CPAD - C)))

    degp = _deg_kernel(dst2)
    dis, xs = _tc1(degp, x)
    p1 = _agg128(src1, dst1, xs)
    t2s = _tc2(p1, xs, dis, W1, b1, W2)
    p2 = _agg32(src2, dst2, t2s)
    t3s = _tc3(p2, t2s, dis, b2, W3p)
    p3 = _agg16(src2, dst2, t3s)
    return _tc4(p3, t3s, dis, b3)
